# baseline (speedup 1.0000x reference)
import functools

import jax
import jax.numpy as jnp
from jax import lax
from jax.experimental import pallas as pl
from jax.experimental.pallas import tpu as pltpu
from jax.experimental.pallas import tpu_sc as plsc

NC = 2
NS = 16
NW = NC * NS
CH = 128
L = 16


def _sc_mesh():
    return plsc.VectorSubcoreMesh(core_axis_name="c", subcore_axis_name="s")


def _make_deg_kernel(n_pad, cpt_a, cpt_loop):
    rps = n_pad // NS

    @functools.partial(
        pl.kernel,
        out_type=jax.ShapeDtypeStruct((NC, n_pad), jnp.float32),
        mesh=_sc_mesh(),
        scratch_types=[
            pltpu.VMEM((cpt_a, CH), jnp.int32),
            pltpu.VMEM((CH,), jnp.float32),
            pltpu.VMEM((rps,), jnp.float32),
            pltpu.VMEM_SHARED((n_pad,), jnp.float32),
        ],
        compiler_params=pltpu.CompilerParams(use_tc_tiling_on_sc=False),
    )
    def deg_kernel(dst_hbm, out_hbm, dst_v, ones_v, zero_v, acc):
        c = lax.axis_index("c")
        s = lax.axis_index("s")
        wid = s * NC + c

        ones16 = jnp.ones((L,), jnp.float32)
        zero16 = jnp.zeros((L,), jnp.float32)
        for j in range(CH // L):
            ones_v[pl.ds(j * L, L)] = ones16

        def zfill(i, carry):
            zero_v[pl.ds(i * L, L)] = zero16
            return carry

        lax.fori_loop(0, rps // L, zfill, 0)
        r0 = s * rps
        pltpu.sync_copy(zero_v, acc.at[pl.ds(r0, rps)])
        plsc.subcore_barrier()

        pltpu.sync_copy(dst_hbm.at[wid], dst_v)

        def body(j, carry):
            pltpu.sync_copy(ones_v, acc.at[dst_v.at[j]], add=True)
            return carry

        lax.fori_loop(0, cpt_loop, body, 0)
        plsc.subcore_barrier()
        pltpu.sync_copy(acc.at[pl.ds(r0, rps)], out_hbm.at[c, pl.ds(r0, rps)])

    return deg_kernel


def _make_hop_kernel(n_pad, cpt_e, feat):
    rps = n_pad // NS
    cpt_a = cpt_e + 2

    @functools.partial(
        pl.kernel,
        out_type=jax.ShapeDtypeStruct((NC, n_pad, feat), jnp.float32),
        mesh=_sc_mesh(),
        scratch_types=[
            pltpu.VMEM((cpt_a, CH), jnp.int32),
            pltpu.VMEM((cpt_a, CH), jnp.int32),
            pltpu.VMEM((CH, feat), jnp.float32),
            pltpu.VMEM((CH, feat), jnp.float32),
            pltpu.VMEM_SHARED((n_pad, feat), jnp.float32),
            pltpu.SemaphoreType.DMA,
            pltpu.SemaphoreType.DMA,
        ],
        compiler_params=pltpu.CompilerParams(use_tc_tiling_on_sc=False),
    )
    def hop_kernel(src_hbm, dst_hbm, g_hbm, out_hbm,
                   src_v, dst_v, r0_v, r1_v, acc, sem_a, sem_b):
        c = lax.axis_index("c")
        s = lax.axis_index("s")
        wid = s * NC + c

        zero16 = jnp.zeros((L,), jnp.float32)

        def zrow(i, carry):
            for j in range(feat // L):
                r0_v[i, pl.ds(j * L, L)] = zero16
            return carry

        lax.fori_loop(0, CH, zrow, 0)
        r0 = s * rps
        for b in range(rps // CH):
            pltpu.sync_copy(r0_v, acc.at[pl.ds(r0 + b * CH, CH)])

        pltpu.sync_copy(src_hbm.at[wid], src_v)
        pltpu.sync_copy(dst_hbm.at[wid], dst_v)
        plsc.subcore_barrier()

        pltpu.async_copy(g_hbm.at[src_v.at[0]], r0_v, sem_a)
        pltpu.async_copy(g_hbm.at[src_v.at[1]], r1_v, sem_b)

        def body(jj, carry):
            j = 2 * jj
            pltpu.make_async_copy(g_hbm.at[src_v.at[j]], r0_v, sem_a).wait()
            pltpu.sync_copy(r0_v, acc.at[dst_v.at[j]], add=True)
            pltpu.async_copy(g_hbm.at[src_v.at[j + 2]], r0_v, sem_a)
            pltpu.make_async_copy(g_hbm.at[src_v.at[j + 1]], r1_v, sem_b).wait()
            pltpu.sync_copy(r1_v, acc.at[dst_v.at[j + 1]], add=True)
            pltpu.async_copy(g_hbm.at[src_v.at[j + 3]], r1_v, sem_b)
            return carry

        lax.fori_loop(0, cpt_e // 2, body, 0)
        pltpu.make_async_copy(g_hbm.at[src_v.at[0]], r0_v, sem_a).wait()
        pltpu.make_async_copy(g_hbm.at[src_v.at[1]], r1_v, sem_b).wait()
        plsc.subcore_barrier()
        pltpu.sync_copy(acc.at[pl.ds(r0, rps)],
                        out_hbm.at[c, pl.ds(r0, rps)])

    return hop_kernel


def _tc_prep(x, w, deg_t):
    n, d = x.shape
    cc = w.shape[0]
    r = 2000

    def body(x_ref, w_ref, dg_ref, g0_ref, s1_ref, s2_ref):
        deg = jnp.sum(dg_ref[...], axis=1, keepdims=True) + 1.0
        dinv = lax.rsqrt(deg)
        y = lax.dot_general(x_ref[...], w_ref[...],
                            (((1,), (1,)), ((), ())),
                            preferred_element_type=jnp.float32)
        g0_ref[...] = y * dinv
        s1_ref[...] = dinv * dinv
        s2_ref[...] = dinv

    return pl.pallas_call(
        body,
        grid=(n // r,),
        in_specs=[
            pl.BlockSpec((r, d), lambda i: (i, 0)),
            pl.BlockSpec((cc, d), lambda i: (0, 0)),
            pl.BlockSpec((r, NC), lambda i: (i, 0)),
        ],
        out_specs=[
            pl.BlockSpec((r, cc), lambda i: (i, 0)),
            pl.BlockSpec((r, 1), lambda i: (i, 0)),
            pl.BlockSpec((r, 1), lambda i: (i, 0)),
        ],
        out_shape=[
            jax.ShapeDtypeStruct((n, cc), jnp.float32),
            jax.ShapeDtypeStruct((n, 1), jnp.float32),
            jax.ShapeDtypeStruct((n, 1), jnp.float32),
        ],
    )(x, w, deg_t)


def _tc_combine(parts, g, scale):
    n, cc = g.shape
    r = 2000

    def body(p_ref, g_ref, s_ref, o_ref):
        p = p_ref[0] + p_ref[1]
        o_ref[...] = s_ref[...] * (p + g_ref[...])

    return pl.pallas_call(
        body,
        grid=(n // r,),
        in_specs=[
            pl.BlockSpec((NC, r, cc), lambda i: (0, i, 0)),
            pl.BlockSpec((r, cc), lambda i: (i, 0)),
            pl.BlockSpec((r, 1), lambda i: (i, 0)),
        ],
        out_specs=pl.BlockSpec((r, cc), lambda i: (i, 0)),
        out_shape=jax.ShapeDtypeStruct((n, cc), jnp.float32),
    )(parts, g, scale)


def kernel(x, edge_index, w):
    n, _ = x.shape
    cc = w.shape[0]
    e = edge_index.shape[1]

    n_pad = -(-(n + 1) // (NS * CH)) * (NS * CH)
    cpt = -(-e // (NW * CH))
    cpt_e = cpt + (cpt % 2)
    cpt_a = cpt_e + 2
    ep = NW * cpt_a * CH

    src = edge_index[0]
    dst = edge_index[1]
    pad = NW * cpt_e * CH - e
    srcp = jnp.concatenate([src, jnp.zeros((pad,), src.dtype)])
    dstp = jnp.concatenate([dst, jnp.full((pad,), n, dst.dtype)])
    src3 = jnp.concatenate(
        [srcp.reshape(NW, cpt_e, CH), jnp.zeros((NW, 2, CH), src.dtype)], axis=1)
    dst3 = jnp.concatenate(
        [dstp.reshape(NW, cpt_e, CH), jnp.full((NW, 2, CH), n, dst.dtype)], axis=1)

    degp = _make_deg_kernel(n_pad, cpt_a, cpt_e)(dst3)
    deg_t = degp.T
    g0, sc1, sc2 = _tc_prep(x, w, deg_t)

    hop = _make_hop_kernel(n_pad, cpt_e, cc)
    p1 = hop(src3, dst3, g0)
    g1 = _tc_combine(p1, g0, sc1)
    p2 = hop(src3, dst3, g1)
    return _tc_combine(p2, g1, sc2)


# speedup vs baseline: 1.4058x; 1.4058x over previous
import functools

import jax
import jax.numpy as jnp
from jax import lax
from jax.experimental import pallas as pl
from jax.experimental.pallas import tpu as pltpu
from jax.experimental.pallas import tpu_sc as plsc

NC = 2
NS = 16
NW = NC * NS
CH = 128
L = 16


def _sc_mesh():
    return plsc.VectorSubcoreMesh(core_axis_name="c", subcore_axis_name="s")


def _make_deg_kernel(n_pad, cpt_a, cpt_loop):
    rps = n_pad // NS

    @functools.partial(
        pl.kernel,
        out_type=jax.ShapeDtypeStruct((NC, n_pad), jnp.float32),
        mesh=_sc_mesh(),
        scratch_types=[
            pltpu.VMEM((cpt_a, CH), jnp.int32),
            pltpu.VMEM((CH,), jnp.float32),
            pltpu.VMEM((rps,), jnp.float32),
            pltpu.VMEM_SHARED((n_pad,), jnp.float32),
        ],
        compiler_params=pltpu.CompilerParams(use_tc_tiling_on_sc=False),
    )
    def deg_kernel(dst_hbm, out_hbm, dst_v, ones_v, zero_v, acc):
        c = lax.axis_index("c")
        s = lax.axis_index("s")
        wid = s * NC + c

        ones16 = jnp.ones((L,), jnp.float32)
        zero16 = jnp.zeros((L,), jnp.float32)
        for j in range(CH // L):
            ones_v[pl.ds(j * L, L)] = ones16

        def zfill(i, carry):
            zero_v[pl.ds(i * L, L)] = zero16
            return carry

        lax.fori_loop(0, rps // L, zfill, 0)
        r0 = s * rps
        pltpu.sync_copy(zero_v, acc.at[pl.ds(r0, rps)])
        plsc.subcore_barrier()

        pltpu.sync_copy(dst_hbm.at[wid], dst_v)

        def body(j, carry):
            pltpu.sync_copy(ones_v, acc.at[dst_v.at[j]], add=True)
            return carry

        lax.fori_loop(0, cpt_loop, body, 0)
        plsc.subcore_barrier()
        pltpu.sync_copy(acc.at[pl.ds(r0, rps)], out_hbm.at[c, pl.ds(r0, rps)])

    return deg_kernel


def _make_hop_kernel(n_pad, cpt_e, feat):
    rps = n_pad // NS
    KB = 4

    @functools.partial(
        pl.kernel,
        out_type=jax.ShapeDtypeStruct((NC, n_pad, feat), jnp.float32),
        mesh=_sc_mesh(),
        scratch_types=[
            pltpu.VMEM((cpt_e, CH), jnp.int32),
            pltpu.VMEM((cpt_e, CH), jnp.int32),
            pltpu.VMEM((KB * CH, feat), jnp.float32),
            pltpu.VMEM_SHARED((n_pad, feat), jnp.float32),
            pltpu.SemaphoreType.DMA,
            pltpu.SemaphoreType.DMA,
        ],
        compiler_params=pltpu.CompilerParams(use_tc_tiling_on_sc=False),
    )
    def hop_kernel(src_hbm, dst_hbm, g_hbm, out_hbm,
                   src_v, dst_v, rows_v, acc, sem_a, sem_b):
        c = lax.axis_index("c")
        s = lax.axis_index("s")
        wid = s * NC + c

        zero16 = jnp.zeros((L,), jnp.float32)

        def zrow(i, carry):
            for j in range(feat // L):
                rows_v[i, pl.ds(j * L, L)] = zero16
            return carry

        lax.fori_loop(0, CH, zrow, 0)
        r0 = s * rps
        for b in range(rps // CH):
            pltpu.sync_copy(rows_v.at[pl.ds(0, CH)],
                            acc.at[pl.ds(r0 + b * CH, CH)])

        pltpu.sync_copy(src_hbm.at[wid], src_v)
        pltpu.sync_copy(dst_hbm.at[wid], dst_v)
        plsc.subcore_barrier()

        def body(jj, carry):
            j = KB * jj
            for k in range(KB):
                pltpu.async_copy(g_hbm.at[src_v.at[j + k]],
                                 rows_v.at[pl.ds(k * CH, CH)], sem_a)
            for k in range(KB):
                pltpu.make_async_copy(g_hbm.at[src_v.at[j + k]],
                                      rows_v.at[pl.ds(k * CH, CH)],
                                      sem_a).wait()
            for k in range(KB):
                pltpu.async_copy(rows_v.at[pl.ds(k * CH, CH)],
                                 acc.at[dst_v.at[j + k]], sem_b, add=True)
            for k in range(KB):
                pltpu.make_async_copy(rows_v.at[pl.ds(k * CH, CH)],
                                      acc.at[dst_v.at[j + k]],
                                      sem_b).wait()
            return carry

        lax.fori_loop(0, cpt_e // KB, body, 0)
        plsc.subcore_barrier()
        pltpu.sync_copy(acc.at[pl.ds(r0, rps)],
                        out_hbm.at[c, pl.ds(r0, rps)])

    return hop_kernel


def _tc_prep(x, w, deg_t):
    n, d = x.shape
    cc = w.shape[0]
    r = 2000

    def body(x_ref, w_ref, dg_ref, g0_ref, s1_ref, s2_ref):
        deg = jnp.sum(dg_ref[...], axis=1, keepdims=True) + 1.0
        dinv = lax.rsqrt(deg)
        y = lax.dot_general(x_ref[...], w_ref[...],
                            (((1,), (1,)), ((), ())),
                            preferred_element_type=jnp.float32)
        g0_ref[...] = y * dinv
        s1_ref[...] = dinv * dinv
        s2_ref[...] = dinv

    return pl.pallas_call(
        body,
        grid=(n // r,),
        in_specs=[
            pl.BlockSpec((r, d), lambda i: (i, 0)),
            pl.BlockSpec((cc, d), lambda i: (0, 0)),
            pl.BlockSpec((r, NC), lambda i: (i, 0)),
        ],
        out_specs=[
            pl.BlockSpec((r, cc), lambda i: (i, 0)),
            pl.BlockSpec((r, 1), lambda i: (i, 0)),
            pl.BlockSpec((r, 1), lambda i: (i, 0)),
        ],
        out_shape=[
            jax.ShapeDtypeStruct((n, cc), jnp.float32),
            jax.ShapeDtypeStruct((n, 1), jnp.float32),
            jax.ShapeDtypeStruct((n, 1), jnp.float32),
        ],
    )(x, w, deg_t)


def _tc_combine(parts, g, scale):
    n, cc = g.shape
    r = 2000

    def body(p_ref, g_ref, s_ref, o_ref):
        p = p_ref[0] + p_ref[1]
        o_ref[...] = s_ref[...] * (p + g_ref[...])

    return pl.pallas_call(
        body,
        grid=(n // r,),
        in_specs=[
            pl.BlockSpec((NC, r, cc), lambda i: (0, i, 0)),
            pl.BlockSpec((r, cc), lambda i: (i, 0)),
            pl.BlockSpec((r, 1), lambda i: (i, 0)),
        ],
        out_specs=pl.BlockSpec((r, cc), lambda i: (i, 0)),
        out_shape=jax.ShapeDtypeStruct((n, cc), jnp.float32),
    )(parts, g, scale)


def kernel(x, edge_index, w):
    n, _ = x.shape
    cc = w.shape[0]
    e = edge_index.shape[1]

    n_pad = -(-(n + 1) // (NS * CH)) * (NS * CH)
    cpt = -(-e // (NW * CH))
    cpt_e = -(-cpt // 4) * 4

    src = edge_index[0]
    dst = edge_index[1]
    pad = NW * cpt_e * CH - e
    srcp = jnp.concatenate([src, jnp.zeros((pad,), src.dtype)])
    dstp = jnp.concatenate([dst, jnp.full((pad,), n, dst.dtype)])
    src3 = srcp.reshape(NW, cpt_e, CH)
    dst3 = dstp.reshape(NW, cpt_e, CH)

    degp = _make_deg_kernel(n_pad, cpt_e, cpt_e)(dst3)
    deg_t = degp.T
    g0, sc1, sc2 = _tc_prep(x, w, deg_t)

    hop = _make_hop_kernel(n_pad, cpt_e, cc)
    p1 = hop(src3, dst3, g0)
    g1 = _tc_combine(p1, g0, sc1)
    p2 = hop(src3, dst3, g1)
    return _tc_combine(p2, g1, sc2)


# speedup vs baseline: 2.6611x; 1.8930x over previous
import functools

import jax
import jax.numpy as jnp
from jax import lax
from jax.experimental import pallas as pl
from jax.experimental.pallas import tpu as pltpu
from jax.experimental.pallas import tpu_sc as plsc

NC = 2
NS = 16
NW = NC * NS
CH = 128
L = 16


def _sc_mesh():
    return plsc.VectorSubcoreMesh(core_axis_name="c", subcore_axis_name="s")


def _make_deg_kernel(n_pad, cpt_a, cpt_loop):
    rps = n_pad // NS

    @functools.partial(
        pl.kernel,
        out_type=jax.ShapeDtypeStruct((NC, n_pad), jnp.float32),
        mesh=_sc_mesh(),
        scratch_types=[
            pltpu.VMEM((cpt_a, CH), jnp.int32),
            pltpu.VMEM((CH,), jnp.float32),
            pltpu.VMEM((rps,), jnp.float32),
            pltpu.VMEM_SHARED((n_pad,), jnp.float32),
        ],
        compiler_params=pltpu.CompilerParams(use_tc_tiling_on_sc=False),
    )
    def deg_kernel(dst_hbm, out_hbm, dst_v, ones_v, zero_v, acc):
        c = lax.axis_index("c")
        s = lax.axis_index("s")
        wid = s * NC + c

        ones16 = jnp.ones((L,), jnp.float32)
        zero16 = jnp.zeros((L,), jnp.float32)
        for j in range(CH // L):
            ones_v[pl.ds(j * L, L)] = ones16

        def zfill(i, carry):
            zero_v[pl.ds(i * L, L)] = zero16
            return carry

        lax.fori_loop(0, rps // L, zfill, 0)
        r0 = s * rps
        pltpu.sync_copy(zero_v, acc.at[pl.ds(r0, rps)])
        plsc.subcore_barrier()

        pltpu.sync_copy(dst_hbm.at[wid], dst_v)

        def body(j, carry):
            pltpu.sync_copy(ones_v, acc.at[dst_v.at[j]], add=True)
            return carry

        lax.fori_loop(0, cpt_loop, body, 0)
        plsc.subcore_barrier()
        pltpu.sync_copy(acc.at[pl.ds(r0, rps)], out_hbm.at[c, pl.ds(r0, rps)])

    return deg_kernel


def _make_hop_kernel(n_pad, cpt_e, feat):
    rps = n_pad // NS

    @functools.partial(
        pl.kernel,
        out_type=jax.ShapeDtypeStruct((NC, n_pad, feat), jnp.float32),
        mesh=_sc_mesh(),
        scratch_types=[
            pltpu.VMEM((cpt_e, CH), jnp.int32),
            pltpu.VMEM((cpt_e, CH), jnp.int32),
            pltpu.VMEM((CH, feat), jnp.float32),
            pltpu.VMEM_SHARED((n_pad, feat), jnp.float32),
            pltpu.VMEM_SHARED((n_pad, feat), jnp.float32),
            pltpu.SemaphoreType.DMA,
        ],
        compiler_params=pltpu.CompilerParams(use_tc_tiling_on_sc=False),
    )
    def hop_kernel(src_hbm, dst_hbm, g_hbm, out_hbm,
                   src_v, dst_v, rows_v, acc, g_sh, sem):
        c = lax.axis_index("c")
        s = lax.axis_index("s")
        wid = s * NC + c
        n = g_hbm.shape[0]

        zero16 = jnp.zeros((L,), jnp.float32)

        def zrow(i, carry):
            for j in range(feat // L):
                rows_v[i, pl.ds(j * L, L)] = zero16
            return carry

        lax.fori_loop(0, CH, zrow, 0)
        r0 = s * rps
        for b in range(rps // CH):
            pltpu.sync_copy(rows_v, acc.at[pl.ds(r0 + b * CH, CH)])

        gps = n // NS
        pltpu.sync_copy(g_hbm.at[pl.ds(s * gps, gps)],
                        g_sh.at[pl.ds(s * gps, gps)])

        pltpu.sync_copy(src_hbm.at[wid], src_v)
        pltpu.sync_copy(dst_hbm.at[wid], dst_v)
        plsc.subcore_barrier()

        def body(j, carry):
            pltpu.async_copy(g_sh.at[src_v.at[j]], rows_v, sem).wait()
            pltpu.sync_copy(rows_v, acc.at[dst_v.at[j]], add=True)
            return carry

        lax.fori_loop(0, cpt_e, body, 0)
        plsc.subcore_barrier()
        pltpu.sync_copy(acc.at[pl.ds(r0, rps)],
                        out_hbm.at[c, pl.ds(r0, rps)])

    return hop_kernel


def _tc_prep(x, w, deg_t):
    n, d = x.shape
    cc = w.shape[0]
    r = 2000

    def body(x_ref, w_ref, dg_ref, g0_ref, s1_ref, s2_ref):
        deg = jnp.sum(dg_ref[...], axis=1, keepdims=True) + 1.0
        dinv = lax.rsqrt(deg)
        y = lax.dot_general(x_ref[...], w_ref[...],
                            (((1,), (1,)), ((), ())),
                            preferred_element_type=jnp.float32)
        g0_ref[...] = y * dinv
        s1_ref[...] = dinv * dinv
        s2_ref[...] = dinv

    return pl.pallas_call(
        body,
        grid=(n // r,),
        in_specs=[
            pl.BlockSpec((r, d), lambda i: (i, 0)),
            pl.BlockSpec((cc, d), lambda i: (0, 0)),
            pl.BlockSpec((r, NC), lambda i: (i, 0)),
        ],
        out_specs=[
            pl.BlockSpec((r, cc), lambda i: (i, 0)),
            pl.BlockSpec((r, 1), lambda i: (i, 0)),
            pl.BlockSpec((r, 1), lambda i: (i, 0)),
        ],
        out_shape=[
            jax.ShapeDtypeStruct((n, cc), jnp.float32),
            jax.ShapeDtypeStruct((n, 1), jnp.float32),
            jax.ShapeDtypeStruct((n, 1), jnp.float32),
        ],
    )(x, w, deg_t)


def _tc_combine(parts, g, scale):
    n, cc = g.shape
    r = 2000

    def body(p_ref, g_ref, s_ref, o_ref):
        p = p_ref[0] + p_ref[1]
        o_ref[...] = s_ref[...] * (p + g_ref[...])

    return pl.pallas_call(
        body,
        grid=(n // r,),
        in_specs=[
            pl.BlockSpec((NC, r, cc), lambda i: (0, i, 0)),
            pl.BlockSpec((r, cc), lambda i: (i, 0)),
            pl.BlockSpec((r, 1), lambda i: (i, 0)),
        ],
        out_specs=pl.BlockSpec((r, cc), lambda i: (i, 0)),
        out_shape=jax.ShapeDtypeStruct((n, cc), jnp.float32),
    )(parts, g, scale)


def kernel(x, edge_index, w):
    n, _ = x.shape
    cc = w.shape[0]
    e = edge_index.shape[1]

    n_pad = -(-(n + 1) // (NS * CH)) * (NS * CH)
    cpt = -(-e // (NW * CH))
    cpt_e = -(-cpt // 4) * 4

    src = edge_index[0]
    dst = edge_index[1]
    pad = NW * cpt_e * CH - e
    srcp = jnp.concatenate([src, jnp.zeros((pad,), src.dtype)])
    dstp = jnp.concatenate([dst, jnp.full((pad,), n, dst.dtype)])
    src3 = srcp.reshape(NW, cpt_e, CH)
    dst3 = dstp.reshape(NW, cpt_e, CH)

    degp = _make_deg_kernel(n_pad, cpt_e, cpt_e)(dst3)
    deg_t = degp.T
    g0, sc1, sc2 = _tc_prep(x, w, deg_t)

    hop = _make_hop_kernel(n_pad, cpt_e, cc)
    p1 = hop(src3, dst3, g0)
    g1 = _tc_combine(p1, g0, sc1)
    p2 = hop(src3, dst3, g1)
    return _tc_combine(p2, g1, sc2)


# speedup vs baseline: 3.2886x; 1.2358x over previous
import functools

import jax
import jax.numpy as jnp
from jax import lax
from jax.experimental import pallas as pl
from jax.experimental.pallas import tpu as pltpu
from jax.experimental.pallas import tpu_sc as plsc

NC = 2
NS = 16
NW = NC * NS
CH = 128
L = 16


def _sc_mesh():
    return plsc.VectorSubcoreMesh(core_axis_name="c", subcore_axis_name="s")


def _make_deg_kernel(n_pad, cpt_a, cpt_loop):
    rps = n_pad // NS

    @functools.partial(
        pl.kernel,
        out_type=jax.ShapeDtypeStruct((NC, n_pad), jnp.float32),
        mesh=_sc_mesh(),
        scratch_types=[
            pltpu.VMEM((cpt_a, CH), jnp.int32),
            pltpu.VMEM((CH,), jnp.float32),
            pltpu.VMEM((rps,), jnp.float32),
            pltpu.VMEM_SHARED((n_pad,), jnp.float32),
        ],
        compiler_params=pltpu.CompilerParams(use_tc_tiling_on_sc=False),
    )
    def deg_kernel(dst_hbm, out_hbm, dst_v, ones_v, zero_v, acc):
        c = lax.axis_index("c")
        s = lax.axis_index("s")
        wid = s * NC + c

        ones16 = jnp.ones((L,), jnp.float32)
        zero16 = jnp.zeros((L,), jnp.float32)
        for j in range(CH // L):
            ones_v[pl.ds(j * L, L)] = ones16

        def zfill(i, carry):
            zero_v[pl.ds(i * L, L)] = zero16
            return carry

        lax.fori_loop(0, rps // L, zfill, 0)
        r0 = s * rps
        pltpu.sync_copy(zero_v, acc.at[pl.ds(r0, rps)])
        plsc.subcore_barrier()

        pltpu.sync_copy(dst_hbm.at[wid], dst_v)

        def body(j, carry):
            pltpu.sync_copy(ones_v, acc.at[dst_v.at[j]], add=True)
            return carry

        lax.fori_loop(0, cpt_loop, body, 0)
        plsc.subcore_barrier()
        pltpu.sync_copy(acc.at[pl.ds(r0, rps)], out_hbm.at[c, pl.ds(r0, rps)])

    return deg_kernel


def _make_hop_kernel(n_pad, cpt_e, feat):
    rps = n_pad // NS

    @functools.partial(
        pl.kernel,
        out_type=jax.ShapeDtypeStruct((NC, n_pad, feat), jnp.float32),
        mesh=_sc_mesh(),
        scratch_types=[
            pltpu.VMEM((cpt_e, CH), jnp.int32),
            pltpu.VMEM((cpt_e, CH), jnp.int32),
            pltpu.VMEM((CH, feat), jnp.float32),
            pltpu.VMEM((CH, feat), jnp.float32),
            pltpu.VMEM_SHARED((n_pad, feat), jnp.float32),
            pltpu.VMEM_SHARED((n_pad, feat), jnp.float32),
            pltpu.SemaphoreType.DMA,
            pltpu.SemaphoreType.DMA,
        ],
        compiler_params=pltpu.CompilerParams(use_tc_tiling_on_sc=False),
    )
    def hop_kernel(src_hbm, dst_hbm, g_hbm, out_hbm,
                   src_v, dst_v, rows_v, r1_v, acc, g_sh, sem, sem_b):
        c = lax.axis_index("c")
        s = lax.axis_index("s")
        wid = s * NC + c
        n = g_hbm.shape[0]

        zero16 = jnp.zeros((L,), jnp.float32)

        def zrow(i, carry):
            for j in range(feat // L):
                rows_v[i, pl.ds(j * L, L)] = zero16
            return carry

        lax.fori_loop(0, CH, zrow, 0)
        r0 = s * rps
        for b in range(rps // CH):
            pltpu.sync_copy(rows_v, acc.at[pl.ds(r0 + b * CH, CH)])

        gps = n // NS
        pltpu.sync_copy(g_hbm.at[pl.ds(s * gps, gps)],
                        g_sh.at[pl.ds(s * gps, gps)])

        pltpu.sync_copy(src_hbm.at[wid], src_v)
        pltpu.sync_copy(dst_hbm.at[wid], dst_v)
        plsc.subcore_barrier()

        pltpu.async_copy(g_sh.at[src_v.at[0]], rows_v, sem)
        pltpu.async_copy(g_sh.at[src_v.at[1]], r1_v, sem_b)

        def body(jj, carry):
            j = 2 * jj
            pltpu.make_async_copy(g_sh.at[src_v.at[j]], rows_v, sem).wait()
            pltpu.sync_copy(rows_v, acc.at[dst_v.at[j]], add=True)

            @pl.when(j + 2 < cpt_e)
            def _():
                pltpu.async_copy(g_sh.at[src_v.at[j + 2]], rows_v, sem)

            pltpu.make_async_copy(g_sh.at[src_v.at[j + 1]], r1_v, sem_b).wait()
            pltpu.sync_copy(r1_v, acc.at[dst_v.at[j + 1]], add=True)

            @pl.when(j + 3 < cpt_e)
            def _():
                pltpu.async_copy(g_sh.at[src_v.at[j + 3]], r1_v, sem_b)

            return carry

        lax.fori_loop(0, cpt_e // 2, body, 0)
        plsc.subcore_barrier()
        pltpu.sync_copy(acc.at[pl.ds(r0, rps)],
                        out_hbm.at[c, pl.ds(r0, rps)])

    return hop_kernel


def _tc_prep(x, w, deg_t):
    n, d = x.shape
    cc = w.shape[0]
    r = 2000

    def body(x_ref, w_ref, dg_ref, g0_ref, s1_ref, s2_ref):
        deg = jnp.sum(dg_ref[...], axis=1, keepdims=True) + 1.0
        dinv = lax.rsqrt(deg)
        y = lax.dot_general(x_ref[...], w_ref[...],
                            (((1,), (1,)), ((), ())),
                            preferred_element_type=jnp.float32)
        g0_ref[...] = y * dinv
        s1_ref[...] = dinv * dinv
        s2_ref[...] = dinv

    return pl.pallas_call(
        body,
        grid=(n // r,),
        in_specs=[
            pl.BlockSpec((r, d), lambda i: (i, 0)),
            pl.BlockSpec((cc, d), lambda i: (0, 0)),
            pl.BlockSpec((r, NC), lambda i: (i, 0)),
        ],
        out_specs=[
            pl.BlockSpec((r, cc), lambda i: (i, 0)),
            pl.BlockSpec((r, 1), lambda i: (i, 0)),
            pl.BlockSpec((r, 1), lambda i: (i, 0)),
        ],
        out_shape=[
            jax.ShapeDtypeStruct((n, cc), jnp.float32),
            jax.ShapeDtypeStruct((n, 1), jnp.float32),
            jax.ShapeDtypeStruct((n, 1), jnp.float32),
        ],
    )(x, w, deg_t)


def _tc_combine(parts, g, scale):
    n, cc = g.shape
    r = 2000

    def body(p_ref, g_ref, s_ref, o_ref):
        p = p_ref[0] + p_ref[1]
        o_ref[...] = s_ref[...] * (p + g_ref[...])

    return pl.pallas_call(
        body,
        grid=(n // r,),
        in_specs=[
            pl.BlockSpec((NC, r, cc), lambda i: (0, i, 0)),
            pl.BlockSpec((r, cc), lambda i: (i, 0)),
            pl.BlockSpec((r, 1), lambda i: (i, 0)),
        ],
        out_specs=pl.BlockSpec((r, cc), lambda i: (i, 0)),
        out_shape=jax.ShapeDtypeStruct((n, cc), jnp.float32),
    )(parts, g, scale)


def kernel(x, edge_index, w):
    n, _ = x.shape
    cc = w.shape[0]
    e = edge_index.shape[1]

    n_pad = -(-(n + 1) // (NS * CH)) * (NS * CH)
    cpt = -(-e // (NW * CH))
    cpt_e = -(-cpt // 4) * 4

    src = edge_index[0]
    dst = edge_index[1]
    pad = NW * cpt_e * CH - e
    srcp = jnp.concatenate([src, jnp.zeros((pad,), src.dtype)])
    dstp = jnp.concatenate([dst, jnp.full((pad,), n, dst.dtype)])
    src3 = srcp.reshape(NW, cpt_e, CH)
    dst3 = dstp.reshape(NW, cpt_e, CH)

    degp = _make_deg_kernel(n_pad, cpt_e, cpt_e)(dst3)
    deg_t = degp.T
    g0, sc1, sc2 = _tc_prep(x, w, deg_t)

    hop = _make_hop_kernel(n_pad, cpt_e, cc)
    p1 = hop(src3, dst3, g0)
    g1 = _tc_combine(p1, g0, sc1)
    p2 = hop(src3, dst3, g1)
    return _tc_combine(p2, g1, sc2)
